# baseline (speedup 1.0000x reference)
import jax
import jax.numpy as jnp
from jax.experimental import pallas as pl
from jax.experimental.pallas import tpu as pltpu


def _pool_body(batch_ref, x_ref, psum_ref, aug_ref, oh_ref, *, tn, ch, nb, c, tps):
    k = pl.program_id(1)
    ca = aug_ref.shape[1]

    @pl.when(k == 0)
    def _init():
        psum_ref[...] = jnp.zeros_like(psum_ref)
        aug_ref[:, c:] = jnp.zeros((tn, ca - c), jnp.bfloat16)
        aug_ref[:, c:c + 1] = jnp.ones((tn, 1), jnp.bfloat16)

    aug_ref[:, :c] = x_ref[...].astype(jnp.bfloat16)

    gid = jax.lax.broadcasted_iota(jnp.int32, (nb, ch), 0)
    for j in range(tn // ch):
        seg = batch_ref[0, :, j * ch:(j + 1) * ch]
        oh_ref[:, j * ch:(j + 1) * ch] = (gid == seg).astype(jnp.bfloat16)

    psum_ref[...] += jax.lax.dot_general(
        aug_ref[...], oh_ref[...],
        dimension_numbers=(((0,), (1,)), ((), ())),
        preferred_element_type=jnp.float32)


def _combine_body(psum_ref, w_ref, bias_ref, o_ref, *, c):
    s = jnp.sum(psum_ref[...], axis=0)
    pooled = s[:c, :] / jnp.maximum(s[c:c + 1, :], 1.0)
    y = jax.lax.dot_general(
        pooled, w_ref[...],
        dimension_numbers=(((0,), (1,)), ((), ())),
        preferred_element_type=jnp.float32)
    o_ref[...] = jnp.maximum(y + bias_ref[...], 0.0)


def _mean_pool_mlp(x, batch, weight, bias, num_graphs, tn, ch, splits=2):
    n, c = x.shape
    h = weight.shape[0]
    assert n % (tn * splits) == 0 and tn % ch == 0
    n_tiles = n // tn
    tps = n_tiles // splits
    ca = ((c + 1 + 7) // 8) * 8

    batch3 = batch.astype(jnp.int32).reshape(n_tiles, 1, tn)
    bias2 = bias.astype(jnp.float32).reshape(1, h)
    w = weight.astype(jnp.float32)

    import functools
    psum = pl.pallas_call(
        functools.partial(_pool_body, tn=tn, ch=ch, nb=num_graphs, c=c, tps=tps),
        out_shape=jax.ShapeDtypeStruct((splits, ca, num_graphs), jnp.float32),
        grid=(splits, tps),
        in_specs=[
            pl.BlockSpec((1, 1, tn), lambda s, k: (s * tps + k, 0, 0)),
            pl.BlockSpec((tn, c), lambda s, k: (s * tps + k, 0)),
        ],
        out_specs=pl.BlockSpec((None, ca, num_graphs), lambda s, k: (s, 0, 0)),
        scratch_shapes=[pltpu.VMEM((tn, ca), jnp.bfloat16),
                        pltpu.VMEM((num_graphs, tn), jnp.bfloat16)],
        compiler_params=pltpu.CompilerParams(
            dimension_semantics=("parallel", "arbitrary"),
            vmem_limit_bytes=56 * 1024 * 1024),
    )(batch3, x)

    bb = num_graphs // splits
    out = pl.pallas_call(
        functools.partial(_combine_body, c=c),
        out_shape=jax.ShapeDtypeStruct((num_graphs, h), jnp.float32),
        grid=(splits,),
        in_specs=[
            pl.BlockSpec((splits, ca, bb), lambda i: (0, 0, i)),
            pl.BlockSpec((h, c), lambda i: (0, 0)),
            pl.BlockSpec((1, h), lambda i: (0, 0)),
        ],
        out_specs=pl.BlockSpec((bb, h), lambda i: (i, 0)),
        compiler_params=pltpu.CompilerParams(
            dimension_semantics=("parallel",),
            vmem_limit_bytes=32 * 1024 * 1024),
    )(psum, w, bias2)
    return out


def kernel(x, batch, weight, bias):
    return _mean_pool_mlp(x, batch, weight, bias, 1024, 8192, 2048, splits=4)


# speedup vs baseline: 1.0347x; 1.0347x over previous
import jax
import jax.numpy as jnp
from jax.experimental import pallas as pl
from jax.experimental.pallas import tpu as pltpu


def _pool_body(batch_ref, x_ref, psum_ref, aug_ref, oh_ref, *, tn, ch, nb, c, tps):
    k = pl.program_id(1)
    ca = aug_ref.shape[1]

    @pl.when(k == 0)
    def _init():
        psum_ref[...] = jnp.zeros_like(psum_ref)
        aug_ref[:, c:] = jnp.zeros((tn, ca - c), jnp.bfloat16)
        aug_ref[:, c:c + 1] = jnp.ones((tn, 1), jnp.bfloat16)

    aug_ref[:, :c] = x_ref[...].astype(jnp.bfloat16)

    gid = jax.lax.broadcasted_iota(jnp.int32, (nb, ch), 0)
    for j in range(tn // ch):
        seg = batch_ref[0, :, j * ch:(j + 1) * ch]
        oh_ref[:, j * ch:(j + 1) * ch] = (gid == seg).astype(jnp.bfloat16)

    psum_ref[...] += jax.lax.dot_general(
        aug_ref[...], oh_ref[...],
        dimension_numbers=(((0,), (1,)), ((), ())),
        preferred_element_type=jnp.float32)


def _combine_body(psum_ref, w_ref, bias_ref, o_ref, *, c):
    s = jnp.sum(psum_ref[...], axis=0)
    pooled = s[:c, :] / jnp.maximum(s[c:c + 1, :], 1.0)
    y = jax.lax.dot_general(
        pooled, w_ref[...],
        dimension_numbers=(((0,), (1,)), ((), ())),
        preferred_element_type=jnp.float32)
    o_ref[...] = jnp.maximum(y + bias_ref[...], 0.0)


def _mean_pool_mlp(x, batch, weight, bias, num_graphs, tn, ch, splits=2):
    n, c = x.shape
    h = weight.shape[0]
    assert n % (tn * splits) == 0 and tn % ch == 0
    n_tiles = n // tn
    tps = n_tiles // splits
    ca = ((c + 1 + 7) // 8) * 8

    batch3 = batch.astype(jnp.int32).reshape(n_tiles, 1, tn)
    bias2 = bias.astype(jnp.float32).reshape(1, h)
    w = weight.astype(jnp.float32)

    import functools
    psum = pl.pallas_call(
        functools.partial(_pool_body, tn=tn, ch=ch, nb=num_graphs, c=c, tps=tps),
        out_shape=jax.ShapeDtypeStruct((splits, ca, num_graphs), jnp.float32),
        grid=(splits, tps),
        in_specs=[
            pl.BlockSpec((1, 1, tn), lambda s, k: (s * tps + k, 0, 0)),
            pl.BlockSpec((tn, c), lambda s, k: (s * tps + k, 0)),
        ],
        out_specs=pl.BlockSpec((None, ca, num_graphs), lambda s, k: (s, 0, 0)),
        scratch_shapes=[pltpu.VMEM((tn, ca), jnp.bfloat16),
                        pltpu.VMEM((num_graphs, tn), jnp.bfloat16)],
        compiler_params=pltpu.CompilerParams(
            dimension_semantics=("parallel", "arbitrary"),
            vmem_limit_bytes=56 * 1024 * 1024),
    )(batch3, x)

    bb = num_graphs // splits
    out = pl.pallas_call(
        functools.partial(_combine_body, c=c),
        out_shape=jax.ShapeDtypeStruct((num_graphs, h), jnp.float32),
        grid=(splits,),
        in_specs=[
            pl.BlockSpec((splits, ca, bb), lambda i: (0, 0, i)),
            pl.BlockSpec((h, c), lambda i: (0, 0)),
            pl.BlockSpec((1, h), lambda i: (0, 0)),
        ],
        out_specs=pl.BlockSpec((bb, h), lambda i: (i, 0)),
        compiler_params=pltpu.CompilerParams(
            dimension_semantics=("parallel",),
            vmem_limit_bytes=32 * 1024 * 1024),
    )(psum, w, bias2)
    return out


def kernel(x, batch, weight, bias):
    return _mean_pool_mlp(x, batch, weight, bias, 1024, 8192, 2048, splits=1)


# speedup vs baseline: 1.0864x; 1.0500x over previous
import functools
import jax
import jax.numpy as jnp
from jax.experimental import pallas as pl
from jax.experimental.pallas import tpu as pltpu


def _body(batch_ref, x_ref, w_ref, bias_ref, o_ref, psum_ref, *,
          tn, ch, nb, c, n_tiles):
    k = pl.program_id(0)

    @pl.when(k == 0)
    def _init():
        psum_ref[...] = jnp.zeros_like(psum_ref)

    gid = jax.lax.broadcasted_iota(jnp.int32, (nb, ch), 0)
    ones = jnp.ones((ch, 8), jnp.bfloat16)
    for j in range(tn // ch):
        xb = x_ref[pl.ds(j * ch, ch), :].astype(jnp.bfloat16)
        aug = jnp.concatenate([xb, ones], axis=1)
        seg = batch_ref[0, :, j * ch:(j + 1) * ch]
        oh = (gid == seg).astype(jnp.bfloat16)
        psum_ref[...] += jax.lax.dot_general(
            aug, oh,
            dimension_numbers=(((0,), (1,)), ((), ())),
            preferred_element_type=jnp.float32)

    @pl.when(k == n_tiles - 1)
    def _finalize():
        s = psum_ref[...]
        pooled = s[:c, :] / jnp.maximum(s[c:c + 1, :], 1.0)
        y = jax.lax.dot_general(
            pooled, w_ref[...],
            dimension_numbers=(((0,), (1,)), ((), ())),
            preferred_element_type=jnp.float32)
        o_ref[...] = jnp.maximum(y + bias_ref[...], 0.0)


def _mean_pool_mlp(x, batch, weight, bias, num_graphs, tn, ch):
    n, c = x.shape
    h = weight.shape[0]
    assert n % tn == 0 and tn % ch == 0
    n_tiles = n // tn

    batch3 = batch.astype(jnp.int32).reshape(n_tiles, 1, tn)
    bias2 = bias.astype(jnp.float32).reshape(1, h)
    w = weight.astype(jnp.float32)

    out = pl.pallas_call(
        functools.partial(_body, tn=tn, ch=ch, nb=num_graphs, c=c,
                          n_tiles=n_tiles),
        out_shape=jax.ShapeDtypeStruct((num_graphs, h), jnp.float32),
        grid=(n_tiles,),
        in_specs=[
            pl.BlockSpec((1, 1, tn), lambda k: (k, 0, 0)),
            pl.BlockSpec((tn, c), lambda k: (k, 0)),
            pl.BlockSpec((h, c), lambda k: (0, 0)),
            pl.BlockSpec((1, h), lambda k: (0, 0)),
        ],
        out_specs=pl.BlockSpec((num_graphs, h), lambda k: (0, 0)),
        scratch_shapes=[pltpu.VMEM((c + 8, num_graphs), jnp.float32)],
        compiler_params=pltpu.CompilerParams(
            dimension_semantics=("arbitrary",),
            vmem_limit_bytes=56 * 1024 * 1024),
    )(batch3, x, w, bias2)
    return out


def kernel(x, batch, weight, bias):
    return _mean_pool_mlp(x, batch, weight, bias, 1024, 8192, 2048)


# speedup vs baseline: 1.4993x; 1.3800x over previous
import functools
import jax
import jax.numpy as jnp
from jax.experimental import pallas as pl
from jax.experimental.pallas import tpu as pltpu


def _body(batch_ref, x_ref, w_ref, bias_ref, o_ref, psum_ref, *,
          tn, ch, nb, c, n_tiles):
    k = pl.program_id(0)

    @pl.when(k == 0)
    def _init():
        psum_ref[...] = jnp.zeros_like(psum_ref)

    gid = jax.lax.broadcasted_iota(jnp.int32, (ch, nb), 1)
    ones = jnp.ones((ch, 8), jnp.bfloat16)
    seg_col = jnp.transpose(batch_ref[0])
    for j in range(tn // ch):
        xb = x_ref[pl.ds(j * ch, ch), :].astype(jnp.bfloat16)
        aug = jnp.concatenate([xb, ones], axis=1)
        sc = seg_col[j * ch:(j + 1) * ch, :]
        oh = (sc == gid).astype(jnp.bfloat16)
        psum_ref[...] += jax.lax.dot_general(
            aug, oh,
            dimension_numbers=(((0,), (0,)), ((), ())),
            preferred_element_type=jnp.float32)

    @pl.when(k == n_tiles - 1)
    def _finalize():
        s = psum_ref[...]
        pooled = s[:c, :] / jnp.maximum(s[c:c + 1, :], 1.0)
        y = jax.lax.dot_general(
            pooled, w_ref[...],
            dimension_numbers=(((0,), (1,)), ((), ())),
            preferred_element_type=jnp.float32)
        o_ref[...] = jnp.maximum(y + bias_ref[...], 0.0)


def _mean_pool_mlp(x, batch, weight, bias, num_graphs, tn, ch):
    n, c = x.shape
    h = weight.shape[0]
    assert n % tn == 0 and tn % ch == 0
    n_tiles = n // tn

    batch3 = batch.astype(jnp.int32).reshape(n_tiles, 1, tn)
    bias2 = bias.astype(jnp.float32).reshape(1, h)
    w = weight.astype(jnp.float32)

    out = pl.pallas_call(
        functools.partial(_body, tn=tn, ch=ch, nb=num_graphs, c=c,
                          n_tiles=n_tiles),
        out_shape=jax.ShapeDtypeStruct((num_graphs, h), jnp.float32),
        grid=(n_tiles,),
        in_specs=[
            pl.BlockSpec((1, 1, tn), lambda k: (k, 0, 0)),
            pl.BlockSpec((tn, c), lambda k: (k, 0)),
            pl.BlockSpec((h, c), lambda k: (0, 0)),
            pl.BlockSpec((1, h), lambda k: (0, 0)),
        ],
        out_specs=pl.BlockSpec((num_graphs, h), lambda k: (0, 0)),
        scratch_shapes=[pltpu.VMEM((c + 8, num_graphs), jnp.float32)],
        compiler_params=pltpu.CompilerParams(
            dimension_semantics=("arbitrary",),
            vmem_limit_bytes=56 * 1024 * 1024),
    )(batch3, x, w, bias2)
    return out


def kernel(x, batch, weight, bias):
    return _mean_pool_mlp(x, batch, weight, bias, 1024, 8192, 2048)
